# Initial kernel scaffold; baseline (speedup 1.0000x reference)
import jax
import jax.numpy as jnp
from jax.experimental import pallas as pl


def kernel(s, emb_w, lin_w, lin_b):
    raise NotImplementedError("write your pallas kernel here")


# speedup vs baseline: 16.7339x; 16.7339x over previous
import functools

import jax
import jax.numpy as jnp
from jax import lax
from jax.experimental import pallas as pl
from jax.experimental.pallas import tpu as pltpu
from jax.experimental.pallas import tpu_sc as plsc

VOCAB = 100000
EMBED_DIM = 128
BATCH = 16384
HIST_LEN = 50
OUT_DIM = 2

PADW = 16
VBLK = 4000

NC, NS = 2, 16
NW = NC * NS
BPW = BATCH // NW
UNROLL = 8


def _proj_body(emb_ref, w_ref, b_ref, out_ref):
    out_ref[...] = (
        jnp.dot(emb_ref[...], w_ref[...], preferred_element_type=jnp.float32)
        + b_ref[...]
    )


def _project_table(emb_w, w_pad, b_pad):
    return pl.pallas_call(
        _proj_body,
        grid=(VOCAB // VBLK,),
        in_specs=[
            pl.BlockSpec((VBLK, EMBED_DIM), lambda i: (i, 0)),
            pl.BlockSpec((EMBED_DIM, PADW), lambda i: (0, 0)),
            pl.BlockSpec((1, PADW), lambda i: (0, 0)),
        ],
        out_specs=pl.BlockSpec((VBLK, PADW), lambda i: (i, 0)),
        out_shape=jax.ShapeDtypeStruct((VOCAB, PADW), jnp.float32),
    )(emb_w, w_pad, b_pad)


@functools.partial(
    pl.kernel,
    out_type=jax.ShapeDtypeStruct((BATCH, PADW), jnp.float32),
    mesh=plsc.VectorSubcoreMesh(core_axis_name="c", subcore_axis_name="s"),
    scratch_types=[
        pltpu.VMEM((HIST_LEN, BPW), jnp.int32),
        pltpu.VMEM((BPW, PADW), jnp.float32),
        pltpu.VMEM((BPW, PADW), jnp.float32),
        pltpu.SemaphoreType.DMA,
    ],
    compiler_params=pltpu.CompilerParams(use_tc_tiling_on_sc=False),
)
def _sc_pool(proj_hbm, st_hbm, out_hbm, idx_v, slab_v, acc_v, sem):
    wid = lax.axis_index("s") * NC + lax.axis_index("c")
    base = wid * BPW
    pltpu.sync_copy(st_hbm.at[:, pl.ds(base, BPW)], idx_v)
    pltpu.async_copy(proj_hbm.at[idx_v.at[0]], acc_v, sem).wait()

    def l_body(l, carry):
        pltpu.async_copy(proj_hbm.at[idx_v.at[l]], slab_v, sem).wait()

        def k_body(k, c):
            for u in range(UNROLL):
                r = k * UNROLL + u
                plsc.addupdate(acc_v.at[r], slab_v[r])
            return c

        return lax.fori_loop(0, BPW // UNROLL, k_body, carry)

    lax.fori_loop(1, HIST_LEN, l_body, 0)
    pltpu.sync_copy(acc_v, out_hbm.at[pl.ds(base, BPW), :])


def kernel(s, emb_w, lin_w, lin_b):
    s32 = s.astype(jnp.int32)
    w_pad = jnp.zeros((EMBED_DIM, PADW), jnp.float32).at[:, :OUT_DIM].set(lin_w.T)
    b_pad = jnp.zeros((1, PADW), jnp.float32).at[:, :OUT_DIM].set(lin_b)
    proj = _project_table(emb_w, w_pad, b_pad)
    s_t = s32.T
    out_pad = _sc_pool(proj, s_t)
    return out_pad[:, :OUT_DIM]


# speedup vs baseline: 23.7789x; 1.4210x over previous
import functools

import jax
import jax.numpy as jnp
from jax import lax
from jax.experimental import pallas as pl
from jax.experimental.pallas import tpu as pltpu
from jax.experimental.pallas import tpu_sc as plsc

VOCAB = 100000
EMBED_DIM = 128
BATCH = 16384
HIST_LEN = 50
OUT_DIM = 2

PADW = 16
VBLK = 4000

NC, NS = 2, 16
NW = NC * NS
BPW = BATCH // NW
GRP = BPW // 16


def _rtne_bf16_hi(u):
    return (u + 0x7FFF + ((u >> 16) & 1)) >> 16


def _proj_body(emb_ref, w_ref, b_ref, out_ref):
    p = (
        jnp.dot(emb_ref[...], w_ref[...], preferred_element_type=jnp.float32)
        + b_ref[...]
    )
    u = lax.bitcast_convert_type(p, jnp.uint32)
    r = _rtne_bf16_hi(u)
    word = r[:, 0:1] | (r[:, 1:2] << 16)
    out_ref[...] = lax.bitcast_convert_type(word, jnp.int32)


def _project_table(emb_w, w_pad, b_pad):
    return pl.pallas_call(
        _proj_body,
        grid=(VOCAB // VBLK,),
        in_specs=[
            pl.BlockSpec((VBLK, EMBED_DIM), lambda i: (i, 0)),
            pl.BlockSpec((EMBED_DIM, PADW), lambda i: (0, 0)),
            pl.BlockSpec((1, PADW), lambda i: (0, 0)),
        ],
        out_specs=pl.BlockSpec((VBLK, 1), lambda i: (i, 0)),
        out_shape=jax.ShapeDtypeStruct((VOCAB, 1), jnp.int32),
    )(emb_w, w_pad, b_pad)


@functools.partial(
    pl.kernel,
    out_type=jax.ShapeDtypeStruct((BATCH * OUT_DIM,), jnp.float32),
    mesh=plsc.VectorSubcoreMesh(core_axis_name="c", subcore_axis_name="s"),
    scratch_types=[
        pltpu.VMEM((VOCAB,), jnp.int32),
        pltpu.VMEM((BPW, HIST_LEN), jnp.int32),
        pltpu.VMEM((BPW * OUT_DIM,), jnp.float32),
    ],
    compiler_params=pltpu.CompilerParams(
        use_tc_tiling_on_sc=False, needs_layout_passes=False
    ),
)
def _sc_pool(proj_hbm, s_hbm, out_hbm, table_v, s_v, out_v):
    wid = lax.axis_index("s") * NC + lax.axis_index("c")
    base = wid * BPW
    pltpu.sync_copy(proj_hbm, table_v)
    pltpu.sync_copy(s_hbm.at[pl.ds(base, BPW), :], s_v)

    lane = lax.iota(jnp.int32, 16)
    zero = jnp.zeros((16,), jnp.float32)
    mask_hi = jnp.full((16,), jnp.int32(-65536))

    def g_body(g, carry):
        idx_b = g * 16 + lane
        acc0, acc1 = zero, zero
        for l in range(HIST_LEN):
            idx_l = jnp.full((16,), l, jnp.int32)
            v = plsc.load_gather(s_v, [idx_b, idx_l])
            w = plsc.load_gather(table_v, [v])
            acc0 = acc0 + plsc.bitcast(w << 16, jnp.float32)
            acc1 = acc1 + plsc.bitcast(w & mask_hi, jnp.float32)
        pos = g * 32 + lane * 2
        plsc.store_scatter(out_v, [pos], acc0)
        plsc.store_scatter(out_v, [pos + 1], acc1)
        return carry

    lax.fori_loop(0, GRP, g_body, 0)
    pltpu.sync_copy(out_v, out_hbm.at[pl.ds(base * OUT_DIM, BPW * OUT_DIM)])


def kernel(s, emb_w, lin_w, lin_b):
    s32 = s.astype(jnp.int32)
    w_pad = jnp.zeros((EMBED_DIM, PADW), jnp.float32).at[:, :OUT_DIM].set(lin_w.T)
    b_pad = jnp.zeros((1, PADW), jnp.float32).at[:, :OUT_DIM].set(lin_b)
    proj = _project_table(emb_w, w_pad, b_pad).reshape(VOCAB)
    out_flat = _sc_pool(proj, s32)
    return out_flat.reshape(BATCH, OUT_DIM)


# speedup vs baseline: 25.7643x; 1.0835x over previous
import functools

import jax
import jax.numpy as jnp
from jax import lax
from jax.experimental import pallas as pl
from jax.experimental.pallas import tpu as pltpu
from jax.experimental.pallas import tpu_sc as plsc

VOCAB = 100000
EMBED_DIM = 128
BATCH = 16384
HIST_LEN = 50
OUT_DIM = 2

PADW = 16
VBLK = 4000

NC, NS = 2, 16
NW = NC * NS
BPW = BATCH // NW
GRP = BPW // 16


def _rtne_bf16_hi(u):
    return (u + 0x7FFF + ((u >> 16) & 1)) >> 16


def _proj_body(emb_ref, w_ref, b_ref, out_ref):
    p = (
        jnp.dot(emb_ref[...], w_ref[...], preferred_element_type=jnp.float32)
        + b_ref[...]
    )
    u = lax.bitcast_convert_type(p, jnp.uint32)
    r = _rtne_bf16_hi(u)
    word = r[:, 0:1] | (r[:, 1:2] << 16)
    out_ref[...] = lax.bitcast_convert_type(word, jnp.int32)


def _project_table(emb_w, w_pad, b_pad):
    return pl.pallas_call(
        _proj_body,
        grid=(VOCAB // VBLK,),
        in_specs=[
            pl.BlockSpec((VBLK, EMBED_DIM), lambda i: (i, 0)),
            pl.BlockSpec((EMBED_DIM, PADW), lambda i: (0, 0)),
            pl.BlockSpec((1, PADW), lambda i: (0, 0)),
        ],
        out_specs=pl.BlockSpec((VBLK, 1), lambda i: (i, 0)),
        out_shape=jax.ShapeDtypeStruct((VOCAB, 1), jnp.int32),
    )(emb_w, w_pad, b_pad)


@functools.partial(
    pl.kernel,
    out_type=jax.ShapeDtypeStruct((BATCH, OUT_DIM), jnp.float32),
    mesh=plsc.VectorSubcoreMesh(core_axis_name="c", subcore_axis_name="s"),
    scratch_types=[
        pltpu.VMEM((VOCAB,), jnp.int32),
        pltpu.VMEM((BPW * HIST_LEN,), jnp.int32),
        pltpu.VMEM((BPW, OUT_DIM), jnp.float32),
    ],
    compiler_params=pltpu.CompilerParams(
        use_tc_tiling_on_sc=False, needs_layout_passes=False
    ),
)
def _sc_pool(proj_hbm, s_hbm, out_hbm, table_v, s_v, out_v):
    wid = lax.axis_index("s") * NC + lax.axis_index("c")
    base = wid * BPW
    pltpu.sync_copy(proj_hbm, table_v)
    pltpu.sync_copy(s_hbm.at[pl.ds(base * HIST_LEN, BPW * HIST_LEN)], s_v)

    lane = lax.iota(jnp.int32, 16)
    zeros_i = jnp.zeros((16,), jnp.int32)
    zero = jnp.zeros((16,), jnp.float32)
    mask_hi = jnp.full((16,), jnp.int32(-65536))

    def g_body(g, carry):
        idx_b = g * 16 + lane
        idx_flat = idx_b * HIST_LEN
        acc0, acc1 = zero, zero
        for l in range(HIST_LEN):
            v = plsc.load_gather(s_v, [idx_flat + l])
            w = plsc.load_gather(table_v, [v])
            acc0 = acc0 + plsc.bitcast(w << 16, jnp.float32)
            acc1 = acc1 + plsc.bitcast(w & mask_hi, jnp.float32)
        plsc.store_scatter(out_v, [idx_b, zeros_i], acc0)
        plsc.store_scatter(out_v, [idx_b, zeros_i + 1], acc1)
        return carry

    lax.fori_loop(0, GRP, g_body, 0)
    pltpu.sync_copy(out_v, out_hbm.at[pl.ds(base, BPW), :])


def kernel(s, emb_w, lin_w, lin_b):
    s32 = s.astype(jnp.int32)
    w_pad = jnp.zeros((EMBED_DIM, PADW), jnp.float32).at[:, :OUT_DIM].set(lin_w.T)
    b_pad = jnp.zeros((1, PADW), jnp.float32).at[:, :OUT_DIM].set(lin_b)
    proj = _project_table(emb_w, w_pad, b_pad).reshape(VOCAB)
    return _sc_pool(proj, s32.reshape(BATCH * HIST_LEN))


# speedup vs baseline: 35.1582x; 1.3646x over previous
import functools

import jax
import jax.numpy as jnp
from jax import lax
from jax.experimental import pallas as pl
from jax.experimental.pallas import tpu as pltpu
from jax.experimental.pallas import tpu_sc as plsc

VOCAB = 100000
EMBED_DIM = 128
BATCH = 16384
HIST_LEN = 50
OUT_DIM = 2

PADW = 16
VBLK = 4096
VGRID = -(-VOCAB // VBLK)

NC, NS = 2, 16
NW = NC * NS
BPW = BATCH // NW
GRP = BPW // 16


def _rtne_bf16_hi(u):
    return (u + 0x7FFF + ((u >> 16) & 1)) >> 16


def _proj_body(emb_ref, w_ref, b_ref, out_ref):
    pT = (
        lax.dot_general(
            w_ref[...],
            emb_ref[...],
            (((1,), (1,)), ((), ())),
            preferred_element_type=jnp.float32,
        )
        + b_ref[...]
    )
    u = lax.bitcast_convert_type(pT, jnp.uint32)
    r = _rtne_bf16_hi(u)
    word = r[0:1, :] | (r[1:2, :] << 16)
    out_ref[...] = lax.bitcast_convert_type(word, jnp.int32)


def _project_table(emb_w, w_pad, b_pad):
    return pl.pallas_call(
        _proj_body,
        grid=(VGRID,),
        in_specs=[
            pl.BlockSpec((VBLK, EMBED_DIM), lambda i: (i, 0)),
            pl.BlockSpec((PADW, EMBED_DIM), lambda i: (0, 0)),
            pl.BlockSpec((PADW, 1), lambda i: (0, 0)),
        ],
        out_specs=pl.BlockSpec((1, VBLK), lambda i: (0, i)),
        out_shape=jax.ShapeDtypeStruct((1, VOCAB), jnp.int32),
    )(emb_w, w_pad, b_pad)


@functools.partial(
    pl.kernel,
    out_type=jax.ShapeDtypeStruct((BATCH, OUT_DIM), jnp.float32),
    mesh=plsc.VectorSubcoreMesh(core_axis_name="c", subcore_axis_name="s"),
    scratch_types=[
        pltpu.VMEM((VOCAB,), jnp.int32),
        pltpu.VMEM((BPW * HIST_LEN,), jnp.int32),
        pltpu.VMEM((BPW, OUT_DIM), jnp.float32),
    ],
    compiler_params=pltpu.CompilerParams(
        use_tc_tiling_on_sc=False, needs_layout_passes=False
    ),
)
def _sc_pool(proj_hbm, s_hbm, out_hbm, table_v, s_v, out_v):
    wid = lax.axis_index("s") * NC + lax.axis_index("c")
    base = wid * BPW
    pltpu.sync_copy(proj_hbm.at[0, :], table_v)
    pltpu.sync_copy(s_hbm.at[pl.ds(base * HIST_LEN, BPW * HIST_LEN)], s_v)

    lane = lax.iota(jnp.int32, 16)
    zeros_i = jnp.zeros((16,), jnp.int32)
    zero = jnp.zeros((16,), jnp.float32)
    mask_hi = jnp.full((16,), jnp.int32(-65536))

    def g_body(g, carry):
        idx_b = g * 16 + lane
        idx_flat = idx_b * HIST_LEN
        acc0, acc1 = zero, zero
        for l in range(HIST_LEN):
            v = plsc.load_gather(s_v, [idx_flat + l])
            w = plsc.load_gather(table_v, [v])
            acc0 = acc0 + plsc.bitcast(w << 16, jnp.float32)
            acc1 = acc1 + plsc.bitcast(w & mask_hi, jnp.float32)
        plsc.store_scatter(out_v, [idx_b, zeros_i], acc0)
        plsc.store_scatter(out_v, [idx_b, zeros_i + 1], acc1)
        return carry

    lax.fori_loop(0, GRP, g_body, 0)
    pltpu.sync_copy(out_v, out_hbm.at[pl.ds(base, BPW), :])


def kernel(s, emb_w, lin_w, lin_b):
    s32 = s.astype(jnp.int32)
    w_pad = jnp.zeros((PADW, EMBED_DIM), jnp.float32).at[:OUT_DIM, :].set(lin_w)
    b_pad = jnp.zeros((PADW, 1), jnp.float32).at[:OUT_DIM, 0].set(lin_b)
    proj = _project_table(emb_w, w_pad, b_pad)
    return _sc_pool(proj, s32.reshape(BATCH * HIST_LEN))
